# baseline (device time: 91748 ns/iter reference)
import functools

import jax
import jax.numpy as jnp
from jax import lax
from jax.experimental import pallas as pl
from jax.experimental.pallas import tpu as pltpu

T = 1024
D = 1024
E = 8
NZ = 2
TS = T // NZ
EL = E // NZ
F = 2048
FB = 1024
NF = F // FB

_sem_signal = getattr(pl, "semaphore_signal", None) or pltpu.semaphore_signal
_sem_wait = getattr(pl, "semaphore_wait", None) or pltpu.semaphore_wait
_CompilerParams = getattr(pltpu, "CompilerParams", None) or pltpu.TPUCompilerParams


def _partner():
    my_x = lax.axis_index("x")
    my_y = lax.axis_index("y")
    my_z = lax.axis_index("z")
    return my_z, (my_x, my_y, 1 - my_z)


def _pair_barrier(partner):
    barrier = pltpu.get_barrier_semaphore()
    _sem_signal(barrier, inc=1, device_id=partner,
                device_id_type=pl.DeviceIdType.MESH)
    _sem_wait(barrier, 1)


def _dispatch_body(x_ref, r_ref, xf_ref, cc_ref,
                   xs_ref, ro_ref, cs_ref, send_sems, recv_sems):
    my_z, partner = _partner()
    pz = 1 - my_z

    _pair_barrier(partner)

    xs_ref[...] = x_ref[...].astype(jnp.bfloat16)
    rdma_x = pltpu.make_async_remote_copy(
        src_ref=xs_ref,
        dst_ref=xf_ref.at[pl.ds(my_z * TS, TS), :],
        send_sem=send_sems.at[0],
        recv_sem=recv_sems.at[0],
        device_id=partner,
        device_id_type=pl.DeviceIdType.MESH,
    )
    rdma_x.start()

    rdma_r = pltpu.make_async_remote_copy(
        src_ref=r_ref,
        dst_ref=ro_ref,
        send_sem=send_sems.at[1],
        recv_sem=recv_sems.at[1],
        device_id=partner,
        device_id_type=pl.DeviceIdType.MESH,
    )
    rdma_r.start()

    xf_ref[pl.ds(my_z * TS, TS), :] = xs_ref[...]

    rdma_r.wait()

    x32 = x_ref[...]
    g_m = lax.dot_general(x32, r_ref[...], (((1,), (0,)), ((), ())),
                          precision=lax.Precision.HIGHEST,
                          preferred_element_type=jnp.float32)
    g_o = lax.dot_general(x32, ro_ref[...], (((1,), (0,)), ((), ())),
                          precision=lax.Precision.HIGHEST,
                          preferred_element_type=jnp.float32)
    gates = jnp.where(my_z == 0,
                      jnp.concatenate([g_m, g_o], axis=1),
                      jnp.concatenate([g_o, g_m], axis=1))

    lanes = lax.broadcasted_iota(jnp.int32, (TS, E), 1)
    neg = jnp.float32(-1e30)
    v1 = jnp.max(gates, axis=1, keepdims=True)
    e1 = jnp.min(jnp.where(gates == v1, lanes, E), axis=1, keepdims=True)
    m1 = lanes == e1
    g2 = jnp.where(m1, neg, gates)
    v2 = jnp.max(g2, axis=1, keepdims=True)
    e2 = jnp.min(jnp.where(g2 == v2, lanes, E), axis=1, keepdims=True)
    m2 = lanes == e2
    b = jnp.exp(v2 - v1)
    den = 1.0 + b
    c = jnp.where(m1, 1.0 / den, 0.0) + jnp.where(m2, b / den, 0.0)

    c_mine = jnp.where(my_z == 0, c[:, :EL], c[:, EL:])
    c_theirs = jnp.where(my_z == 0, c[:, EL:], c[:, :EL])

    cc_ref[pl.ds(my_z * TS, TS), :] = c_mine
    cs_ref[...] = c_theirs
    rdma_c = pltpu.make_async_remote_copy(
        src_ref=cs_ref,
        dst_ref=cc_ref.at[pl.ds(my_z * TS, TS), :],
        send_sem=send_sems.at[2],
        recv_sem=recv_sems.at[2],
        device_id=partner,
        device_id_type=pl.DeviceIdType.MESH,
    )
    rdma_c.start()
    rdma_c.wait()
    rdma_x.wait()


def _dispatch(x, router):
    return pl.pallas_call(
        _dispatch_body,
        out_shape=(
            jax.ShapeDtypeStruct((T, D), jnp.bfloat16),
            jax.ShapeDtypeStruct((T, EL), jnp.float32),
        ),
        in_specs=[
            pl.BlockSpec(memory_space=pltpu.VMEM),
            pl.BlockSpec(memory_space=pltpu.VMEM),
        ],
        out_specs=(
            pl.BlockSpec(memory_space=pltpu.VMEM),
            pl.BlockSpec(memory_space=pltpu.VMEM),
        ),
        scratch_shapes=[
            pltpu.VMEM((TS, D), jnp.bfloat16),
            pltpu.VMEM((D, EL), jnp.float32),
            pltpu.VMEM((TS, EL), jnp.float32),
            pltpu.SemaphoreType.DMA((3,)),
            pltpu.SemaphoreType.DMA((3,)),
        ],
        compiler_params=_CompilerParams(collective_id=0),
    )(x, router)


def _expert_body(xf_ref, cc_ref, w1_ref, w2_ref, out_ref):
    e = pl.program_id(0)
    f = pl.program_id(1)

    @pl.when(jnp.logical_and(e == 0, f == 0))
    def _():
        out_ref[...] = jnp.zeros_like(out_ref)

    xb = xf_ref[...]
    w1b = w1_ref[0].astype(jnp.bfloat16)
    h = jnp.maximum(
        lax.dot(xb, w1b, preferred_element_type=jnp.float32), 0.0
    ).astype(jnp.bfloat16)
    w2b = w2_ref[0].astype(jnp.bfloat16)
    o = lax.dot(h, w2b, preferred_element_type=jnp.float32)

    lanes4 = lax.broadcasted_iota(jnp.int32, (T, EL), 1)
    ccol = jnp.sum(jnp.where(lanes4 == e, cc_ref[...], 0.0),
                   axis=1, keepdims=True)
    out_ref[...] += o * ccol


def _experts(x_full, c_cols, W1, W2):
    return pl.pallas_call(
        _expert_body,
        grid=(EL, NF),
        out_shape=jax.ShapeDtypeStruct((T, D), jnp.float32),
        in_specs=[
            pl.BlockSpec((T, D), lambda e, f: (0, 0)),
            pl.BlockSpec((T, EL), lambda e, f: (0, 0)),
            pl.BlockSpec((1, D, FB), lambda e, f: (e, 0, f)),
            pl.BlockSpec((1, FB, D), lambda e, f: (e, f, 0)),
        ],
        out_specs=pl.BlockSpec((T, D), lambda e, f: (0, 0)),
        compiler_params=_CompilerParams(
            dimension_semantics=("arbitrary", "arbitrary"),
        ),
    )(x_full, c_cols, W1, W2)


def _combine_body(p_ref, out_ref, sb_ref, rb_ref, send_sem, recv_sem):
    my_z, partner = _partner()
    pz = 1 - my_z

    _pair_barrier(partner)

    sb_ref[...] = p_ref[pl.ds(pz * TS, TS), :].astype(jnp.bfloat16)
    rdma = pltpu.make_async_remote_copy(
        src_ref=sb_ref,
        dst_ref=rb_ref,
        send_sem=send_sem,
        recv_sem=recv_sem,
        device_id=partner,
        device_id_type=pl.DeviceIdType.MESH,
    )
    rdma.start()
    rdma.wait()

    out_ref[...] = (p_ref[pl.ds(my_z * TS, TS), :]
                    + rb_ref[...].astype(jnp.float32))


def _combine(partial):
    return pl.pallas_call(
        _combine_body,
        out_shape=jax.ShapeDtypeStruct((TS, D), jnp.float32),
        in_specs=[pl.BlockSpec(memory_space=pltpu.VMEM)],
        out_specs=pl.BlockSpec(memory_space=pltpu.VMEM),
        scratch_shapes=[
            pltpu.VMEM((TS, D), jnp.bfloat16),
            pltpu.VMEM((TS, D), jnp.bfloat16),
            pltpu.SemaphoreType.DMA,
            pltpu.SemaphoreType.DMA,
        ],
        compiler_params=_CompilerParams(collective_id=1),
    )(partial)


def kernel(x, router, W1, W2):
    x_full, c_cols = _dispatch(x, router)
    partial = _experts(x_full, c_cols, W1, W2)
    return _combine(partial)


# device time: 85486 ns/iter; 1.0733x vs baseline; 1.0733x over previous
import jax
import jax.numpy as jnp
from jax import lax
from jax.experimental import pallas as pl
from jax.experimental.pallas import tpu as pltpu

T = 1024
D = 1024
E = 8
NZ = 2
TS = T // NZ
EL = E // NZ
F = 2048
FB = 1024
NF = F // FB

_sem_signal = getattr(pl, "semaphore_signal", None) or pltpu.semaphore_signal
_sem_wait = getattr(pl, "semaphore_wait", None) or pltpu.semaphore_wait
_CompilerParams = getattr(pltpu, "CompilerParams", None) or pltpu.TPUCompilerParams

_RTR, _X, _C, _CB = 0, 1, 2, 3


def _body(x_ref, r_ref, w1_ref, w2_ref, out_ref,
          xs_ref, xp_ref, ro_ref, cm_ref, cs_ref, cp_ref,
          accp_ref, sb_ref, rb_ref, send_sems, recv_sems):
    h = pl.program_id(0)
    e = pl.program_id(1)
    f = pl.program_id(2)

    my_x = lax.axis_index("x")
    my_y = lax.axis_index("y")
    my_z = lax.axis_index("z")
    partner = (my_x, my_y, 1 - my_z)

    def _rdma(src, dst, slot):
        return pltpu.make_async_remote_copy(
            src_ref=src, dst_ref=dst,
            send_sem=send_sems.at[slot], recv_sem=recv_sems.at[slot],
            device_id=partner, device_id_type=pl.DeviceIdType.MESH,
        )

    first = jnp.logical_and(h == 0, jnp.logical_and(e == 0, f == 0))
    pass2 = jnp.logical_and(h == 1, jnp.logical_and(e == 0, f == 0))
    last = jnp.logical_and(h == 1, jnp.logical_and(e == EL - 1, f == NF - 1))

    @pl.when(first)
    def _():
        barrier = pltpu.get_barrier_semaphore()
        _sem_signal(barrier, inc=1, device_id=partner,
                    device_id_type=pl.DeviceIdType.MESH)
        _sem_wait(barrier, 1)

        rdma_r = _rdma(r_ref, ro_ref, _RTR)
        rdma_r.start()

        xs_ref[...] = x_ref[...].astype(jnp.bfloat16)
        _rdma(xs_ref, xp_ref, _X).start()

        out_ref[...] = jnp.zeros_like(out_ref)
        accp_ref[...] = jnp.zeros_like(accp_ref)

        rdma_r.wait()

        x32 = x_ref[...]
        g_m = lax.dot_general(x32, r_ref[...], (((1,), (0,)), ((), ())),
                              precision=lax.Precision.HIGHEST,
                              preferred_element_type=jnp.float32)
        g_o = lax.dot_general(x32, ro_ref[...], (((1,), (0,)), ((), ())),
                              precision=lax.Precision.HIGHEST,
                              preferred_element_type=jnp.float32)
        gates = jnp.where(my_z == 0,
                          jnp.concatenate([g_m, g_o], axis=1),
                          jnp.concatenate([g_o, g_m], axis=1))

        lanes = lax.broadcasted_iota(jnp.int32, (TS, E), 1)
        neg = jnp.float32(-1e30)
        v1 = jnp.max(gates, axis=1, keepdims=True)
        e1 = jnp.min(jnp.where(gates == v1, lanes, E), axis=1, keepdims=True)
        m1 = lanes == e1
        g2 = jnp.where(m1, neg, gates)
        v2 = jnp.max(g2, axis=1, keepdims=True)
        e2 = jnp.min(jnp.where(g2 == v2, lanes, E), axis=1, keepdims=True)
        m2 = lanes == e2
        b = jnp.exp(v2 - v1)
        den = 1.0 + b
        c = (jnp.where(m1, 1.0 / den, 0.0)
             + jnp.where(m2, b / den, 0.0))

        cm_ref[...] = jnp.where(my_z == 0, c[:, :EL], c[:, EL:])
        cs_ref[...] = jnp.where(my_z == 0, c[:, EL:], c[:, :EL])
        _rdma(cs_ref, cp_ref, _C).start()

    @pl.when(pass2)
    def _():
        _rdma(xs_ref, xp_ref, _X).wait()
        _rdma(cs_ref, cp_ref, _C).wait()

    xb = jnp.where(h == 0, xs_ref[...], xp_ref[...])
    cc = jnp.where(h == 0, cm_ref[...], cp_ref[...])

    w1b = w1_ref[0].astype(jnp.bfloat16)
    hid = jnp.maximum(
        lax.dot(xb, w1b, preferred_element_type=jnp.float32), 0.0
    ).astype(jnp.bfloat16)
    w2b = w2_ref[0].astype(jnp.bfloat16)
    o = lax.dot(hid, w2b, preferred_element_type=jnp.float32)

    lanes4 = lax.broadcasted_iota(jnp.int32, (TS, EL), 1)
    ccol = jnp.sum(jnp.where(lanes4 == e, cc, 0.0), axis=1, keepdims=True)
    contrib = o * ccol

    @pl.when(h == 0)
    def _():
        out_ref[...] += contrib

    @pl.when(h == 1)
    def _():
        accp_ref[...] += contrib

    @pl.when(last)
    def _():
        sb_ref[...] = accp_ref[...].astype(jnp.bfloat16)
        rdma_cb = _rdma(sb_ref, rb_ref, _CB)
        rdma_cb.start()
        rdma_cb.wait()
        out_ref[...] += rb_ref[...].astype(jnp.float32)


def kernel(x, router, W1, W2):
    return pl.pallas_call(
        _body,
        grid=(2, EL, NF),
        out_shape=jax.ShapeDtypeStruct((TS, D), jnp.float32),
        in_specs=[
            pl.BlockSpec(memory_space=pltpu.VMEM),
            pl.BlockSpec(memory_space=pltpu.VMEM),
            pl.BlockSpec((1, D, FB), lambda h, e, f: (e, 0, f)),
            pl.BlockSpec((1, FB, D), lambda h, e, f: (e, f, 0)),
        ],
        out_specs=pl.BlockSpec((TS, D), lambda h, e, f: (0, 0)),
        scratch_shapes=[
            pltpu.VMEM((TS, D), jnp.bfloat16),
            pltpu.VMEM((TS, D), jnp.bfloat16),
            pltpu.VMEM((D, EL), jnp.float32),
            pltpu.VMEM((TS, EL), jnp.float32),
            pltpu.VMEM((TS, EL), jnp.float32),
            pltpu.VMEM((TS, EL), jnp.float32),
            pltpu.VMEM((TS, D), jnp.float32),
            pltpu.VMEM((TS, D), jnp.bfloat16),
            pltpu.VMEM((TS, D), jnp.bfloat16),
            pltpu.SemaphoreType.DMA((4,)),
            pltpu.SemaphoreType.DMA((4,)),
        ],
        compiler_params=_CompilerParams(
            collective_id=0,
            dimension_semantics=("arbitrary", "arbitrary", "arbitrary"),
        ),
    )(x, router, W1, W2)


# device time: 78283 ns/iter; 1.1720x vs baseline; 1.0920x over previous
import jax
import jax.numpy as jnp
from jax import lax
from jax.experimental import pallas as pl
from jax.experimental.pallas import tpu as pltpu

T = 1024
D = 1024
E = 8
NZ = 2
TS = T // NZ
EL = E // NZ
F = 2048
FB = 1024
NF = F // FB

_sem_signal = getattr(pl, "semaphore_signal", None) or pltpu.semaphore_signal
_sem_wait = getattr(pl, "semaphore_wait", None) or pltpu.semaphore_wait
_CompilerParams = getattr(pltpu, "CompilerParams", None) or pltpu.TPUCompilerParams

_RTR, _X, _C, _CB = 0, 1, 2, 3


def _body(x_ref, r_ref, w1_ref, w2_ref, out_ref,
          xf_ref, ro_ref, cm_ref, cs_ref, cp_ref, cf_ref,
          pacc_ref, sb_ref, rb_ref, w1c0_ref, w2c0_ref,
          send_sems, recv_sems):
    p = pl.program_id(0)
    f = pl.program_id(1)

    my_x = lax.axis_index("x")
    my_y = lax.axis_index("y")
    my_z = lax.axis_index("z")
    pz = 1 - my_z
    partner = (my_x, my_y, pz)

    def _rdma(src, dst, slot):
        return pltpu.make_async_remote_copy(
            src_ref=src, dst_ref=dst,
            send_sem=send_sems.at[slot], recv_sem=recv_sems.at[slot],
            device_id=partner, device_id_type=pl.DeviceIdType.MESH,
        )

    def _rdma_x():
        return _rdma(xf_ref.at[pl.ds(my_z * TS, TS), :],
                     xf_ref.at[pl.ds(my_z * TS, TS), :], _X)

    first = jnp.logical_and(p == 0, f == 0)
    xwait = jnp.logical_and(p == 1, f == 0)
    send_cb = jnp.logical_and(p == 4, f == NF - 1)
    last = jnp.logical_and(p == 5, f == NF - 1)

    @pl.when(first)
    def _():
        barrier = pltpu.get_barrier_semaphore()
        _sem_signal(barrier, inc=1, device_id=partner,
                    device_id_type=pl.DeviceIdType.MESH)
        _sem_wait(barrier, 1)

        rdma_r = _rdma(r_ref, ro_ref, _RTR)
        rdma_r.start()

        xf_ref[pl.ds(my_z * TS, TS), :] = x_ref[...].astype(jnp.bfloat16)
        _rdma_x().start()

        pacc_ref[...] = jnp.zeros_like(pacc_ref)

        rdma_r.wait()

        x32 = x_ref[...]
        g_m = lax.dot_general(x32, r_ref[...], (((1,), (0,)), ((), ())),
                              precision=lax.Precision.HIGHEST,
                              preferred_element_type=jnp.float32)
        g_o = lax.dot_general(x32, ro_ref[...], (((1,), (0,)), ((), ())),
                              precision=lax.Precision.HIGHEST,
                              preferred_element_type=jnp.float32)
        gates = jnp.where(my_z == 0,
                          jnp.concatenate([g_m, g_o], axis=1),
                          jnp.concatenate([g_o, g_m], axis=1))

        lanes = lax.broadcasted_iota(jnp.int32, (TS, E), 1)
        neg = jnp.float32(-1e30)
        v1 = jnp.max(gates, axis=1, keepdims=True)
        e1 = jnp.min(jnp.where(gates == v1, lanes, E), axis=1, keepdims=True)
        m1 = lanes == e1
        g2 = jnp.where(m1, neg, gates)
        v2 = jnp.max(g2, axis=1, keepdims=True)
        e2 = jnp.min(jnp.where(g2 == v2, lanes, E), axis=1, keepdims=True)
        m2 = lanes == e2
        b = jnp.exp(v2 - v1)
        den = 1.0 + b
        c = (jnp.where(m1, 1.0 / den, 0.0)
             + jnp.where(m2, b / den, 0.0))

        cm_ref[...] = jnp.where(my_z == 0, c[:, :EL], c[:, EL:])
        cs_ref[...] = jnp.where(my_z == 0, c[:, EL:], c[:, :EL])
        _rdma(cs_ref, cp_ref, _C).start()

    @pl.when(xwait)
    def _():
        _rdma_x().wait()
        _rdma(cs_ref, cp_ref, _C).wait()
        cf_ref[pl.ds(my_z * TS, TS), :] = cm_ref[...]
        cf_ref[pl.ds(pz * TS, TS), :] = cp_ref[...]

    lanes4 = lax.broadcasted_iota(jnp.int32, (TS, EL), 1)

    def _half_step(w1c, w2c, expert, is_my):
        off = jnp.where(is_my, my_z, pz) * TS
        xh = xf_ref[pl.ds(off, TS), :]
        w1b = w1c[:, pl.ds(f * FB, FB)]
        hid = jnp.maximum(
            lax.dot(xh, w1b, preferred_element_type=jnp.float32), 0.0
        ).astype(jnp.bfloat16)
        w2b = w2c[pl.ds(f * FB, FB), :]
        o = lax.dot(hid, w2b, preferred_element_type=jnp.float32)
        cc = jnp.where(is_my, cm_ref[...], cp_ref[...])
        ccol = jnp.sum(jnp.where(lanes4 == expert, cc, 0.0),
                       axis=1, keepdims=True)
        pacc_ref[pl.ds(off, TS), :] += o * ccol

    @pl.when(jnp.logical_or(p == 0, p == 4))
    def _():
        @pl.when(p == 0)
        def _():
            w1c0_ref[:, pl.ds(f * FB, FB)] = w1_ref[0].astype(jnp.bfloat16)
            w2c0_ref[pl.ds(f * FB, FB), :] = w2_ref[0].astype(jnp.bfloat16)
        _half_step(w1c0_ref, w2c0_ref, 0, p == 0)

    @pl.when(jnp.logical_or(p == 3, p == 5))
    def _():
        off = jnp.where(p == 5, my_z, pz) * TS
        xh = xf_ref[pl.ds(off, TS), :]
        w1b = w1_ref[0].astype(jnp.bfloat16)
        hid = jnp.maximum(
            lax.dot(xh, w1b, preferred_element_type=jnp.float32), 0.0
        ).astype(jnp.bfloat16)
        w2b = w2_ref[0].astype(jnp.bfloat16)
        o = lax.dot(hid, w2b, preferred_element_type=jnp.float32)
        cc = jnp.where(p == 5, cm_ref[...], cp_ref[...])
        ccol = jnp.sum(jnp.where(lanes4 == EL - 1, cc, 0.0),
                       axis=1, keepdims=True)
        pacc_ref[pl.ds(off, TS), :] += o * ccol

    @pl.when(jnp.logical_or(p == 1, p == 2))
    def _():
        xb = xf_ref[...]
        w1b = w1_ref[0].astype(jnp.bfloat16)
        hid = jnp.maximum(
            lax.dot(xb, w1b, preferred_element_type=jnp.float32), 0.0
        ).astype(jnp.bfloat16)
        w2b = w2_ref[0].astype(jnp.bfloat16)
        o = lax.dot(hid, w2b, preferred_element_type=jnp.float32)

        lanes4t = lax.broadcasted_iota(jnp.int32, (T, EL), 1)
        ccol = jnp.sum(jnp.where(lanes4t == p, cf_ref[...], 0.0),
                       axis=1, keepdims=True)
        pacc_ref[...] += o * ccol

    @pl.when(send_cb)
    def _():
        sb_ref[...] = pacc_ref[pl.ds(pz * TS, TS), :].astype(jnp.bfloat16)
        _rdma(sb_ref, rb_ref, _CB).start()

    @pl.when(last)
    def _():
        rdma_cb = _rdma(sb_ref, rb_ref, _CB)
        rdma_cb.wait()
        out_ref[...] = (pacc_ref[pl.ds(my_z * TS, TS), :]
                        + rb_ref[...].astype(jnp.float32))


def _widx(p, f):
    e_idx = jnp.where(p <= 3, jnp.where(p == 0, 0, p), EL - 1)
    f_idx = jnp.where(p == 4, NF - 1, f)
    return e_idx, f_idx


def kernel(x, router, W1, W2):
    def w1_map(p, f):
        e_idx, f_idx = _widx(p, f)
        return (e_idx, 0, f_idx)

    def w2_map(p, f):
        e_idx, f_idx = _widx(p, f)
        return (e_idx, f_idx, 0)

    return pl.pallas_call(
        _body,
        grid=(6, NF),
        out_shape=jax.ShapeDtypeStruct((TS, D), jnp.float32),
        in_specs=[
            pl.BlockSpec(memory_space=pltpu.VMEM),
            pl.BlockSpec(memory_space=pltpu.VMEM),
            pl.BlockSpec((1, D, FB), w1_map),
            pl.BlockSpec((1, FB, D), w2_map),
        ],
        out_specs=pl.BlockSpec((TS, D), lambda p, f: (0, 0)),
        scratch_shapes=[
            pltpu.VMEM((T, D), jnp.bfloat16),
            pltpu.VMEM((D, EL), jnp.float32),
            pltpu.VMEM((TS, EL), jnp.float32),
            pltpu.VMEM((TS, EL), jnp.float32),
            pltpu.VMEM((TS, EL), jnp.float32),
            pltpu.VMEM((T, EL), jnp.float32),
            pltpu.VMEM((T, D), jnp.float32),
            pltpu.VMEM((TS, D), jnp.bfloat16),
            pltpu.VMEM((TS, D), jnp.bfloat16),
            pltpu.VMEM((D, F), jnp.bfloat16),
            pltpu.VMEM((F, D), jnp.bfloat16),
            pltpu.SemaphoreType.DMA((4,)),
            pltpu.SemaphoreType.DMA((4,)),
        ],
        compiler_params=_CompilerParams(
            collective_id=0,
            dimension_semantics=("arbitrary", "arbitrary"),
            vmem_limit_bytes=63 * 1024 * 1024,
        ),
    )(x, router, W1, W2)
